# baseline (speedup 1.0000x reference)
import functools
import math

import jax
import jax.numpy as jnp
from jax import lax
from jax.experimental import pallas as pl
from jax.experimental.pallas import tpu as pltpu
from jax.experimental.pallas import tpu_sc as plsc

VOCAB = 100000
D = 128
B = 4096
H = 20
HPAD = 24
NC, NS = 2, 16
NW = NC * NS
ROWS_W = B // NW
RPC = 4
NCH = ROWS_W // RPC
GLEN = RPC * H
SLEN = RPC * HPAD
SCALE = float(math.sqrt(float(D)))

_mesh = plsc.VectorSubcoreMesh(core_axis_name="c", subcore_axis_name="s")


@functools.partial(
    pl.kernel,
    out_type=jax.ShapeDtypeStruct((B, H, D), jnp.float32),
    mesh=_mesh,
    scratch_types=[
        pltpu.VMEM((ROWS_W * H,), jnp.int32),
        pltpu.VMEM((GLEN, D), jnp.float32),
        pltpu.VMEM((GLEN, D), jnp.float32),
        pltpu.VMEM((SLEN, D), jnp.float32),
        pltpu.VMEM((SLEN, D), jnp.float32),
        pltpu.SemaphoreType.DMA,
        pltpu.SemaphoreType.DMA,
        pltpu.SemaphoreType.DMA,
        pltpu.SemaphoreType.DMA,
    ],
    compiler_params=pltpu.CompilerParams(use_tc_tiling_on_sc=True),
)
def _embed_gather(idx_hbm, table_hbm, out_hbm, idx_v,
                  g_a, g_b, s_a, s_b, gsem_a, gsem_b, ssem_a, ssem_b):
    gbufs = (g_a, g_b)
    sbufs = (s_a, s_b)
    gsems = (gsem_a, gsem_b)
    ssems = (ssem_a, ssem_b)
    wid = lax.axis_index("s") * NC + lax.axis_index("c")
    b0 = wid * ROWS_W

    pltpu.sync_copy(idx_hbm.at[pl.ds(wid * ROWS_W * H, ROWS_W * H)], idx_v)

    pltpu.async_copy(table_hbm.at[idx_v.at[pl.ds(0, GLEN)]], gbufs[0], gsems[0])

    for j in range(NCH):
        p = j % 2
        gbuf, sbuf = gbufs[p], sbufs[p]
        pltpu.make_async_copy(
            table_hbm.at[idx_v.at[pl.ds(j * GLEN, GLEN)]], gbuf, gsems[p]
        ).wait()
        if j + 1 < NCH:
            pltpu.async_copy(
                table_hbm.at[idx_v.at[pl.ds((j + 1) * GLEN, GLEN)]],
                gbufs[1 - p], gsems[1 - p],
            )
        if j >= 2:
            for br in range(RPC):
                pltpu.make_async_copy(
                    sbuf.at[pl.ds(br * HPAD, H)],
                    out_hbm.at[b0 + (j - 2) * RPC + br],
                    ssems[p],
                ).wait()

        def scale_row(h, _, gbuf=gbuf, sbuf=sbuf):
            for br in range(RPC):
                for q in range(D // 16):
                    sbuf[br * HPAD + h, pl.ds(q * 16, 16)] = (
                        gbuf[br * H + h, pl.ds(q * 16, 16)] * SCALE)
            return 0

        lax.fori_loop(0, H, scale_row, 0)

        for br in range(RPC):
            pltpu.async_copy(
                sbuf.at[pl.ds(br * HPAD, H)],
                out_hbm.at[b0 + j * RPC + br],
                ssems[p],
            )

    for j in (NCH - 2, NCH - 1):
        p = j % 2
        for br in range(RPC):
            pltpu.make_async_copy(
                sbufs[p].at[pl.ds(br * HPAD, H)],
                out_hbm.at[b0 + j * RPC + br],
                ssems[p],
            ).wait()


def kernel(x, input_embedding_table):
    idx = x.astype(jnp.int32).reshape(B * H)
    return _embed_gather(idx, input_embedding_table)
